# baseline (speedup 1.0000x reference)
import functools

import jax
import jax.numpy as jnp
from jax.experimental import pallas as pl
from jax.experimental.pallas import tpu as pltpu

_VMEM_LIMIT = 100 * 1024 * 1024


def _conv_stats_kernel(x_ref, w_ref, y_ref, psum_ref, psq_ref, *, kk, ho, wo):
    pad = (kk - 1) // 2
    xs = jnp.pad(
        x_ref[...].astype(jnp.bfloat16),
        ((0, 0), (pad, pad), (pad, pad), (0, 0)),
    )
    nb = xs.shape[0]
    cols = [
        xs[:, kh:kh + ho, kw:kw + wo, :]
        for kh in range(kk) for kw in range(kk)
    ]
    p = jnp.concatenate(cols, axis=-1).reshape(nb * ho * wo, -1)
    yf = jnp.dot(p, w_ref[...], preferred_element_type=jnp.float32)
    y_ref[...] = yf.astype(y_ref.dtype)
    psum_ref[...] = jnp.sum(yf, axis=0, keepdims=True)[None]
    psq_ref[...] = jnp.sum(yf * yf, axis=0, keepdims=True)[None]


def _bn_relu_kernel(y_ref, scale_ref, shift_ref, o_ref):
    o_ref[...] = jnp.maximum(
        y_ref[...].astype(jnp.float32) * scale_ref[...] + shift_ref[...], 0.0
    )


@functools.partial(jax.jit, static_argnames=())
def kernel(x, w, b, gamma, beta):
    eps = 1e-5
    N, Cin, H, W = x.shape
    Cout = w.shape[0]
    K = w.shape[2]
    Ho, Wo = H, W
    HW = Ho * Wo
    M = N * HW
    KKC = K * K * Cin
    pad = (K - 1) // 2
    Hp, Wp = Ho + 2 * pad, Wo + 2 * pad
    del b

    x_nhwc = jnp.transpose(x, (0, 2, 3, 1))
    w2d = jnp.transpose(w, (2, 3, 1, 0)).reshape(KKC, Cout).astype(jnp.bfloat16)

    nb = 16
    while N % nb:
        nb //= 2
    G = N // nb
    body = functools.partial(_conv_stats_kernel, kk=K, ho=Ho, wo=Wo)
    y2d, psum, psq = pl.pallas_call(
        body,
        out_shape=(
            jax.ShapeDtypeStruct((M, Cout), jnp.bfloat16),
            jax.ShapeDtypeStruct((G, 1, Cout), jnp.float32),
            jax.ShapeDtypeStruct((G, 1, Cout), jnp.float32),
        ),
        grid=(G,),
        in_specs=[
            pl.BlockSpec((nb, Ho, Wo, Cin), lambda i: (i, 0, 0, 0)),
            pl.BlockSpec((KKC, Cout), lambda i: (0, 0)),
        ],
        out_specs=[
            pl.BlockSpec((nb * HW, Cout), lambda i: (i, 0)),
            pl.BlockSpec((1, 1, Cout), lambda i: (i, 0, 0)),
            pl.BlockSpec((1, 1, Cout), lambda i: (i, 0, 0)),
        ],
        compiler_params=pltpu.CompilerParams(
            dimension_semantics=("parallel",),
            vmem_limit_bytes=_VMEM_LIMIT,
        ),
        cost_estimate=pl.CostEstimate(
            flops=2 * M * KKC * Cout,
            transcendentals=0,
            bytes_accessed=2 * M * Cin + 2 * KKC * Cout + 2 * M * Cout,
        ),
    )(x_nhwc, w2d)

    inv_m = 1.0 / float(M)
    mean = jnp.sum(psum, axis=0) * inv_m
    var = jnp.maximum(jnp.sum(psq, axis=0) * inv_m - mean * mean, 0.0)
    g2d = gamma.reshape(1, Cout).astype(jnp.float32)
    b2d = beta.reshape(1, Cout).astype(jnp.float32)
    scale = g2d * jax.lax.rsqrt(var + eps)
    shift = b2d - mean * scale

    tm = 16384
    while M % tm:
        tm //= 2
    out2d = pl.pallas_call(
        _bn_relu_kernel,
        out_shape=jax.ShapeDtypeStruct((M, Cout), jnp.float32),
        grid=(M // tm,),
        in_specs=[
            pl.BlockSpec((tm, Cout), lambda i: (i, 0)),
            pl.BlockSpec((1, Cout), lambda i: (0, 0)),
            pl.BlockSpec((1, Cout), lambda i: (0, 0)),
        ],
        out_specs=pl.BlockSpec((tm, Cout), lambda i: (i, 0)),
        compiler_params=pltpu.CompilerParams(
            dimension_semantics=("parallel",),
            vmem_limit_bytes=_VMEM_LIMIT,
        ),
        cost_estimate=pl.CostEstimate(
            flops=3 * M * Cout,
            transcendentals=0,
            bytes_accessed=6 * M * Cout,
        ),
    )(y2d, scale, shift)

    return jnp.transpose(out2d.reshape(N, Ho, Wo, Cout), (0, 3, 1, 2))


# speedup vs baseline: 1.0518x; 1.0518x over previous
import functools

import jax
import jax.numpy as jnp
from jax.experimental import pallas as pl
from jax.experimental.pallas import tpu as pltpu

_VMEM_LIMIT = 100 * 1024 * 1024


def _conv_stats_kernel(x_ref, w_ref, y_ref, psum_ref, psq_ref, *, kk, ho, wo):
    pad = (kk - 1) // 2
    xs = jnp.pad(
        x_ref[...].astype(jnp.bfloat16),
        ((0, 0), (pad, pad), (pad, pad), (0, 0)),
    )
    nb = xs.shape[0]
    cols = [
        xs[:, kh:kh + ho, kw:kw + wo, :]
        for kh in range(kk) for kw in range(kk)
    ]
    p = jnp.concatenate(cols, axis=-1).reshape(nb * ho * wo, -1)
    yf = jnp.dot(p, w_ref[...], preferred_element_type=jnp.float32)
    y_ref[...] = yf.astype(y_ref.dtype)
    psum_ref[...] = jnp.sum(yf, axis=0, keepdims=True)[None]
    psq_ref[...] = jnp.sum(yf * yf, axis=0, keepdims=True)[None]


def _bn_relu_kernel(y_ref, psum_ref, psq_ref, g_ref, b_ref, o_ref, *, inv_m, eps):
    mean = jnp.sum(psum_ref[...], axis=0) * inv_m
    var = jnp.maximum(jnp.sum(psq_ref[...], axis=0) * inv_m - mean * mean, 0.0)
    scale = g_ref[...] * jax.lax.rsqrt(var + eps)
    shift = b_ref[...] - mean * scale
    o_ref[...] = jnp.maximum(
        y_ref[...].astype(jnp.float32) * scale + shift, 0.0
    )


@functools.partial(jax.jit, static_argnames=())
def kernel(x, w, b, gamma, beta):
    eps = 1e-5
    N, Cin, H, W = x.shape
    Cout = w.shape[0]
    K = w.shape[2]
    Ho, Wo = H, W
    HW = Ho * Wo
    M = N * HW
    KKC = K * K * Cin
    pad = (K - 1) // 2
    Hp, Wp = Ho + 2 * pad, Wo + 2 * pad
    del b

    x_nhwc = jnp.transpose(x, (0, 2, 3, 1))
    w2d = jnp.transpose(w, (2, 3, 1, 0)).reshape(KKC, Cout).astype(jnp.bfloat16)

    nb = 8
    while N % nb:
        nb //= 2
    G = N // nb
    body = functools.partial(_conv_stats_kernel, kk=K, ho=Ho, wo=Wo)
    y2d, psum, psq = pl.pallas_call(
        body,
        out_shape=(
            jax.ShapeDtypeStruct((M, Cout), jnp.bfloat16),
            jax.ShapeDtypeStruct((G, 1, Cout), jnp.float32),
            jax.ShapeDtypeStruct((G, 1, Cout), jnp.float32),
        ),
        grid=(G,),
        in_specs=[
            pl.BlockSpec((nb, Ho, Wo, Cin), lambda i: (i, 0, 0, 0)),
            pl.BlockSpec((KKC, Cout), lambda i: (0, 0)),
        ],
        out_specs=[
            pl.BlockSpec((nb * HW, Cout), lambda i: (i, 0)),
            pl.BlockSpec((1, 1, Cout), lambda i: (i, 0, 0)),
            pl.BlockSpec((1, 1, Cout), lambda i: (i, 0, 0)),
        ],
        compiler_params=pltpu.CompilerParams(
            dimension_semantics=("parallel",),
            vmem_limit_bytes=_VMEM_LIMIT,
        ),
        cost_estimate=pl.CostEstimate(
            flops=2 * M * KKC * Cout,
            transcendentals=0,
            bytes_accessed=2 * M * Cin + 2 * KKC * Cout + 2 * M * Cout,
        ),
    )(x_nhwc, w2d)

    g2d = gamma.reshape(1, Cout).astype(jnp.float32)
    b2d = beta.reshape(1, Cout).astype(jnp.float32)
    tm = 16384
    while M % tm:
        tm //= 2
    body2 = functools.partial(_bn_relu_kernel, inv_m=1.0 / float(M), eps=eps)
    out2d = pl.pallas_call(
        body2,
        out_shape=jax.ShapeDtypeStruct((M, Cout), jnp.float32),
        grid=(M // tm,),
        in_specs=[
            pl.BlockSpec((tm, Cout), lambda i: (i, 0)),
            pl.BlockSpec((G, 1, Cout), lambda i: (0, 0, 0)),
            pl.BlockSpec((G, 1, Cout), lambda i: (0, 0, 0)),
            pl.BlockSpec((1, Cout), lambda i: (0, 0)),
            pl.BlockSpec((1, Cout), lambda i: (0, 0)),
        ],
        out_specs=pl.BlockSpec((tm, Cout), lambda i: (i, 0)),
        compiler_params=pltpu.CompilerParams(
            dimension_semantics=("parallel",),
            vmem_limit_bytes=_VMEM_LIMIT,
        ),
        cost_estimate=pl.CostEstimate(
            flops=3 * M * Cout,
            transcendentals=0,
            bytes_accessed=6 * M * Cout,
        ),
    )(y2d, psum, psq, g2d, b2d)

    return jnp.transpose(out2d.reshape(N, Ho, Wo, Cout), (0, 3, 1, 2))
